# baseline (speedup 1.0000x reference)
import functools

import jax
import jax.numpy as jnp
from jax import lax
from jax.experimental import pallas as pl

N = 4096
D = 256
K = 10
KPAD = 16
NUM_BATCH = 8
MIN_SIZE = 10
CC_ITERS = 25

ROWS_BLK = 256


def _normalize_body(emb_ref, out_ref):
    x = emb_ref[...]
    nrm = jnp.sqrt(jnp.sum(x * x, axis=1, keepdims=True))
    out_ref[...] = x / (nrm + 1e-12)


def _normalize(emb):
    grid = N // ROWS_BLK
    return pl.pallas_call(
        _normalize_body,
        grid=(grid,),
        in_specs=[pl.BlockSpec((ROWS_BLK, D), lambda i: (i, 0))],
        out_specs=pl.BlockSpec((ROWS_BLK, D), lambda i: (i, 0)),
        out_shape=jax.ShapeDtypeStruct((N, D), jnp.float32),
    )(emb)


def _topk_body(a_ref, full_ref, bcol_ref, brow_ref, dst_ref):
    r0 = pl.program_id(0) * ROWS_BLK
    a = a_ref[...]
    full = full_ref[...]
    sim = lax.dot_general(a, full, (((1,), (1,)), ((), ())),
                          preferred_element_type=jnp.float32)
    col = lax.broadcasted_iota(jnp.int32, (ROWS_BLK, N), 1)
    rowg = r0 + lax.broadcasted_iota(jnp.int32, (ROWS_BLK, N), 0)
    same = bcol_ref[...] == brow_ref[...]
    neg = jnp.float32(-jnp.inf)
    masked = jnp.where(same & (col != rowg), sim, neg)

    selfc = r0 + lax.broadcasted_iota(jnp.int32, (ROWS_BLK, KPAD), 0)
    dstblk = selfc
    col16 = lax.broadcasted_iota(jnp.int32, (ROWS_BLK, KPAD), 1)
    for k in range(K):
        m = jnp.max(masked, axis=1, keepdims=True)
        pick = jnp.min(jnp.where(masked == m, col, N), axis=1,
                       keepdims=True)
        hit = col == pick
        t = jnp.max(jnp.where(hit, sim, neg), axis=1, keepdims=True)
        masked = jnp.where(hit, neg, masked)
        keep = t >= 0.0
        dstblk = jnp.where((col16 == k) & keep, pick, dstblk)
    dst_ref[...] = dstblk


def _topk_edges(embn, batch):
    grid = N // ROWS_BLK
    bcol = batch.reshape(N, 1)
    brow = batch.reshape(1, N)
    return pl.pallas_call(
        _topk_body,
        grid=(grid,),
        in_specs=[
            pl.BlockSpec((ROWS_BLK, D), lambda i: (i, 0)),
            pl.BlockSpec((N, D), lambda i: (0, 0)),
            pl.BlockSpec((ROWS_BLK, 1), lambda i: (i, 0)),
            pl.BlockSpec((1, N), lambda i: (0, 0)),
        ],
        out_specs=pl.BlockSpec((ROWS_BLK, KPAD), lambda i: (i, 0)),
        out_shape=jax.ShapeDtypeStruct((N, KPAD), jnp.int32),
    )(embn, embn, bcol, brow)


def kernel(emb, batch):
    embn = _normalize(emb)
    dst = _topk_edges(embn, batch)

    srcf = jnp.repeat(jnp.arange(N, dtype=jnp.int32), KPAD)
    dstf = dst.reshape(-1)
    labels = jnp.arange(N, dtype=jnp.int32)
    for _ in range(CC_ITERS):
        m1 = jax.ops.segment_min(labels[srcf], dstf, num_segments=N)
        m2 = jax.ops.segment_min(labels[dstf], srcf, num_segments=N)
        labels = jnp.minimum(labels, jnp.minimum(m1, m2))
        labels = labels[labels]
    counts = jnp.bincount(labels, length=N)
    valid = counts[labels] >= MIN_SIZE
    seg = jnp.where(valid, labels, N)
    cent = jax.ops.segment_sum(jnp.where(valid[:, None], embn, 0.0), seg,
                               num_segments=N)
    cn = jnp.linalg.norm(cent, axis=1, keepdims=True)
    centroids = cent / jnp.where(cn > 0, cn, 1.0)
    cb = jnp.full((N,), NUM_BATCH, dtype=jnp.int32).at[seg].set(
        batch.astype(jnp.int32), mode='drop')
    order = jnp.argsort(cb)
    return centroids[order], cb[order]


# speedup vs baseline: 15.4556x; 15.4556x over previous
import jax
import jax.numpy as jnp
from jax import lax
from jax.experimental import pallas as pl
from jax.experimental.pallas import tpu as pltpu
from jax.experimental.pallas import tpu_sc as plsc

N = 4096
D = 256
K = 10
KPAD = 16
NUM_BATCH = 8
MIN_SIZE = 10
CC_ITERS = 25

ROWS_BLK = 256
NT = 16
G = N // 16


def _normalize_body(emb_ref, out_ref):
    x = emb_ref[...]
    nrm = jnp.sqrt(jnp.sum(x * x, axis=1, keepdims=True))
    out_ref[...] = x / (nrm + 1e-12)


def _normalize(emb):
    return pl.pallas_call(
        _normalize_body,
        grid=(N // ROWS_BLK,),
        in_specs=[pl.BlockSpec((ROWS_BLK, D), lambda i: (i, 0))],
        out_specs=pl.BlockSpec((ROWS_BLK, D), lambda i: (i, 0)),
        out_shape=jax.ShapeDtypeStruct((N, D), jnp.float32),
    )(emb)


def _topk_body(a_ref, full_ref, bcol_ref, brow_ref, dst_ref):
    r0 = pl.program_id(0) * ROWS_BLK
    a = a_ref[...]
    full = full_ref[...]
    sim = lax.dot_general(a, full, (((1,), (1,)), ((), ())),
                          preferred_element_type=jnp.float32)
    col = lax.broadcasted_iota(jnp.int32, (ROWS_BLK, N), 1)
    rowg = r0 + lax.broadcasted_iota(jnp.int32, (ROWS_BLK, N), 0)
    same = bcol_ref[...] == brow_ref[...]
    neg = jnp.float32(-jnp.inf)
    masked = jnp.where(same & (col != rowg), sim, neg)

    selfc = r0 + lax.broadcasted_iota(jnp.int32, (ROWS_BLK, KPAD), 0)
    dstblk = selfc
    col16 = lax.broadcasted_iota(jnp.int32, (ROWS_BLK, KPAD), 1)
    for k in range(K):
        m = jnp.max(masked, axis=1, keepdims=True)
        pick = jnp.min(jnp.where(masked == m, col, N), axis=1,
                       keepdims=True)
        hit = col == pick
        t = jnp.max(jnp.where(hit, sim, neg), axis=1, keepdims=True)
        masked = jnp.where(hit, neg, masked)
        dstblk = jnp.where((col16 == k) & (t >= 0.0), pick, dstblk)
    dst_ref[...] = dstblk


def _topk_edges(embn, batch):
    return pl.pallas_call(
        _topk_body,
        grid=(N // ROWS_BLK,),
        in_specs=[
            pl.BlockSpec((ROWS_BLK, D), lambda i: (i, 0)),
            pl.BlockSpec((N, D), lambda i: (0, 0)),
            pl.BlockSpec((ROWS_BLK, 1), lambda i: (i, 0)),
            pl.BlockSpec((1, N), lambda i: (0, 0)),
        ],
        out_specs=pl.BlockSpec((ROWS_BLK, KPAD), lambda i: (i, 0)),
        out_shape=jax.ShapeDtypeStruct((N, KPAD), jnp.int32),
    )(embn, embn, batch.reshape(N, 1), batch.reshape(1, N))


def _renorm_body(c_ref, out_ref):
    x = c_ref[...]
    cn = jnp.sqrt(jnp.sum(x * x, axis=1, keepdims=True))
    out_ref[...] = x / jnp.where(cn > 0, cn, 1.0)


def _renorm(cent):
    return pl.pallas_call(
        _renorm_body,
        grid=(N // ROWS_BLK,),
        in_specs=[pl.BlockSpec((ROWS_BLK, D), lambda i: (i, 0))],
        out_specs=pl.BlockSpec((ROWS_BLK, D), lambda i: (i, 0)),
        out_shape=jax.ShapeDtypeStruct((N, D), jnp.float32),
    )(cent)


LO_STEP = N // NT
CSC = 32
NCH = N // CSC
GC = CSC // 16


def _sc_tail_body(embn, dstm, batchi, cent_out, cb_out,
                  labels_sh,
                  dstv, labels_v, batch_v, hist, cent_local, chunk, cbb):
    c = lax.axis_index("c")
    s = lax.axis_index("s")
    lanes = lax.iota(jnp.int32, 16)
    zeros16f = jnp.zeros((16,), jnp.float32)

    @pl.when(c == 0)
    def _core0():
        lo = s * LO_STEP

        def _z(r, _):
            for j in range(D // 16):
                chunk[r, pl.ds(16 * j, 16)] = zeros16f
            return 0
        lax.fori_loop(0, CSC, _z, 0)

        def _zo(i, _):
            pltpu.sync_copy(chunk, cent_out.at[pl.ds(lo + i * CSC, CSC)])
            return 0
        lax.fori_loop(0, LO_STEP // CSC, _zo, 0)

        def _zc(r, _):
            for j in range(D // 16):
                cent_local[r, pl.ds(16 * j, 16)] = zeros16f
            return 0
        lax.fori_loop(0, LO_STEP, _zc, 0)

        def _zh(g, _):
            hist[pl.ds(g * 16, 16)] = jnp.zeros((16,), jnp.int32)
            return 0
        lax.fori_loop(0, G, _zh, 0)

        @pl.when(s == 0)
        def _tile0():
            pltpu.sync_copy(dstm, dstv)

            def _init(g, _):
                labels_v[pl.ds(g * 16, 16)] = g * 16 + lanes
                return 0
            lax.fori_loop(0, G, _init, 0)

            def _sweep(it, _):
                def _m2(g, _2):
                    rows = g * 16 + lanes
                    acc = labels_v[pl.ds(g * 16, 16)]
                    flat = rows * K
                    for k in range(K):
                        dk = plsc.load_gather(dstv, [flat + k])
                        acc = jnp.minimum(acc,
                                          plsc.load_gather(labels_v, [dk]))
                    labels_v[pl.ds(g * 16, 16)] = acc
                    return 0
                lax.fori_loop(0, G, _m2, 0)

                def _m1(g, _2):
                    rows = g * 16 + lanes
                    lab = labels_v[pl.ds(g * 16, 16)]
                    flat = rows * K
                    for k in range(K):
                        dk = plsc.load_gather(dstv, [flat + k])
                        cur = plsc.load_gather(labels_v, [dk])
                        plsc.store_scatter(labels_v, [dk],
                                           jnp.minimum(cur, lab))
                    return 0
                lax.fori_loop(0, G, _m1, 0)

                def _jump(g, _2):
                    lab = labels_v[pl.ds(g * 16, 16)]
                    labels_v[pl.ds(g * 16, 16)] = plsc.load_gather(
                        labels_v, [lab])
                    return 0
                lax.fori_loop(0, G, _jump, 0)
                return 0
            lax.fori_loop(0, CC_ITERS, _sweep, 0)

            pltpu.sync_copy(labels_v, labels_sh)

        plsc.subcore_barrier()

        pltpu.sync_copy(labels_sh, labels_v)
        pltpu.sync_copy(batchi, batch_v)

        def _scan(ci, _):
            pltpu.sync_copy(embn.at[pl.ds(ci * CSC, CSC)], chunk)
            for gq in range(GC):
                lab16 = labels_v[pl.ds(ci * CSC + gq * 16, 16)]
                inr = jnp.where((lab16 >= lo) & (lab16 < lo + LO_STEP),
                                jnp.int32(1), jnp.int32(0))
                for r in range(16):
                    li = jnp.full((16,), lab16[r], jnp.int32)
                    hv = plsc.load_gather(hist, [li])
                    plsc.store_scatter(hist, [li], hv + 1)

                    @pl.when(inr[r] == 1)
                    def _add(lab16=lab16, r=r, gq=gq):
                        lloc = lab16[r] - lo
                        for j in range(D // 16):
                            cent_local[lloc, pl.ds(16 * j, 16)] = (
                                cent_local[lloc, pl.ds(16 * j, 16)]
                                + chunk[gq * 16 + r, pl.ds(16 * j, 16)])
            return 0
        lax.fori_loop(0, NCH, _scan, 0)

        def _basef(g, b):
            p16 = g * 16 + lanes
            vr = jnp.where((labels_v[pl.ds(g * 16, 16)] == p16)
                           & (hist[pl.ds(g * 16, 16)] >= MIN_SIZE),
                           jnp.int32(1), jnp.int32(0))
            return b + jnp.sum(vr)
        base = lax.fori_loop(0, s * (LO_STEP // 16), _basef, jnp.int32(0))

        def _place(g, rk):
            p16 = lo + g * 16 + lanes
            vr = jnp.where((labels_v[pl.ds(lo + g * 16, 16)] == p16)
                           & (hist[pl.ds(lo + g * 16, 16)] >= MIN_SIZE),
                           jnp.int32(1), jnp.int32(0))
            for r in range(16):

                @pl.when(vr[r] == 1)
                def _row(g=g, r=r, rk=rk, vr=vr):
                    rk2 = rk + jnp.sum(jnp.where(lanes < r, vr, 0))
                    pltpu.sync_copy(
                        cent_local.at[pl.ds(g * 16 + r, 1)],
                        cent_out.at[pl.ds(base + rk2, 1)])
            return rk + jnp.sum(vr)
        lax.fori_loop(0, LO_STEP // 16, _place, jnp.int32(0))

        @pl.when(s == 0)
        def _cb():
            def _rh(g, rooth):
                p16 = g * 16 + lanes
                vr = jnp.where((labels_v[pl.ds(g * 16, 16)] == p16)
                               & (hist[pl.ds(g * 16, 16)] >= MIN_SIZE),
                               jnp.int32(1), jnp.int32(0))
                bat16 = batch_v[pl.ds(g * 16, 16)]
                for r in range(16):
                    rooth = rooth + jnp.where(
                        (lanes == bat16[r]) & (vr[r] == 1),
                        jnp.int32(1), jnp.int32(0))
                return rooth
            rooth = lax.fori_loop(0, G, _rh, jnp.zeros((16,), jnp.int32))
            cum = plsc.cumsum(rooth)

            def _fill(o, _):
                def _grp(g, _2):
                    j16 = o * 256 + g * 16 + lanes
                    cb16 = jnp.zeros((16,), jnp.int32)
                    for b in range(NUM_BATCH):
                        cb16 = cb16 + jnp.where(j16 >= cum[b],
                                                jnp.int32(1), jnp.int32(0))
                    cbb[pl.ds(g * 16, 16)] = cb16
                    return 0
                lax.fori_loop(0, 16, _grp, 0)
                pltpu.sync_copy(cbb, cb_out.at[pl.ds(o * 256, 256)])
                return 0
            lax.fori_loop(0, 16, _fill, 0)


def _sc_tail(embn, dst, batch):
    mesh = plsc.VectorSubcoreMesh(core_axis_name="c", subcore_axis_name="s")
    f = pl.kernel(
        _sc_tail_body,
        out_type=[
            jax.ShapeDtypeStruct((N, D), jnp.float32),
            jax.ShapeDtypeStruct((N,), jnp.int32),
        ],
        mesh=mesh,
        compiler_params=pltpu.CompilerParams(needs_layout_passes=False),
        scratch_types=[
            pltpu.VMEM_SHARED((N,), jnp.int32),
            pltpu.VMEM((N * K,), jnp.int32),
            pltpu.VMEM((N,), jnp.int32),
            pltpu.VMEM((N,), jnp.int32),
            pltpu.VMEM((N,), jnp.int32),
            pltpu.VMEM((LO_STEP, D), jnp.float32),
            pltpu.VMEM((CSC, D), jnp.float32),
            pltpu.VMEM((256,), jnp.int32),
        ],
    )
    return f(embn, dst, batch)


def kernel(emb, batch):
    embn = _normalize(emb)
    dst = _topk_edges(embn, batch)
    dst_flat = dst[:, :K].reshape(N * K)
    cent_raw, cb_sorted = _sc_tail(embn, dst_flat, batch)
    centroids = _renorm(cent_raw)
    return centroids, cb_sorted


# speedup vs baseline: 16.7614x; 1.0845x over previous
import jax
import jax.numpy as jnp
from jax import lax
from jax.experimental import pallas as pl
from jax.experimental.pallas import tpu as pltpu
from jax.experimental.pallas import tpu_sc as plsc

N = 4096
D = 256
K = 10
KPAD = 16
NUM_BATCH = 8
MIN_SIZE = 10
CC_ITERS = 25

ROWS_BLK = 256
NT = 16
G = N // 16


def _normalize_body(emb_ref, out_ref):
    x = emb_ref[...]
    nrm = jnp.sqrt(jnp.sum(x * x, axis=1, keepdims=True))
    out_ref[...] = x / (nrm + 1e-12)


def _normalize(emb):
    return pl.pallas_call(
        _normalize_body,
        grid=(N // ROWS_BLK,),
        in_specs=[pl.BlockSpec((ROWS_BLK, D), lambda i: (i, 0))],
        out_specs=pl.BlockSpec((ROWS_BLK, D), lambda i: (i, 0)),
        out_shape=jax.ShapeDtypeStruct((N, D), jnp.float32),
    )(emb)


def _topk_body(a_ref, full_ref, bcol_ref, brow_ref, dst_ref):
    r0 = pl.program_id(0) * ROWS_BLK
    a = a_ref[...]
    full = full_ref[...]
    sim = lax.dot_general(a, full, (((1,), (1,)), ((), ())),
                          preferred_element_type=jnp.float32)
    col = lax.broadcasted_iota(jnp.int32, (ROWS_BLK, N), 1)
    rowg = r0 + lax.broadcasted_iota(jnp.int32, (ROWS_BLK, N), 0)
    same = bcol_ref[...] == brow_ref[...]
    neg = jnp.float32(-jnp.inf)
    masked = jnp.where(same & (col != rowg), sim, neg)

    selfc = r0 + lax.broadcasted_iota(jnp.int32, (ROWS_BLK, KPAD), 0)
    dstblk = selfc
    col16 = lax.broadcasted_iota(jnp.int32, (ROWS_BLK, KPAD), 1)
    for k in range(K):
        m = jnp.max(masked, axis=1, keepdims=True)
        pick = jnp.min(jnp.where(masked == m, col, N), axis=1,
                       keepdims=True)
        hit = col == pick
        t = jnp.max(jnp.where(hit, sim, neg), axis=1, keepdims=True)
        masked = jnp.where(hit, neg, masked)
        dstblk = jnp.where((col16 == k) & (t >= 0.0), pick, dstblk)
    dst_ref[...] = dstblk


def _topk_edges(embn, batch):
    return pl.pallas_call(
        _topk_body,
        grid=(N // ROWS_BLK,),
        in_specs=[
            pl.BlockSpec((ROWS_BLK, D), lambda i: (i, 0)),
            pl.BlockSpec((N, D), lambda i: (0, 0)),
            pl.BlockSpec((ROWS_BLK, 1), lambda i: (i, 0)),
            pl.BlockSpec((1, N), lambda i: (0, 0)),
        ],
        out_specs=pl.BlockSpec((ROWS_BLK, KPAD), lambda i: (i, 0)),
        out_shape=jax.ShapeDtypeStruct((N, KPAD), jnp.int32),
    )(embn, embn, batch.reshape(N, 1), batch.reshape(1, N))


def _renorm_body(c_ref, out_ref):
    x = c_ref[...]
    cn = jnp.sqrt(jnp.sum(x * x, axis=1, keepdims=True))
    out_ref[...] = x / jnp.where(cn > 0, cn, 1.0)


def _renorm(cent):
    return pl.pallas_call(
        _renorm_body,
        grid=(N // ROWS_BLK,),
        in_specs=[pl.BlockSpec((ROWS_BLK, D), lambda i: (i, 0))],
        out_specs=pl.BlockSpec((ROWS_BLK, D), lambda i: (i, 0)),
        out_shape=jax.ShapeDtypeStruct((N, D), jnp.float32),
    )(cent)


LO_STEP = N // NT
CSC = 32
NCH = N // CSC
GC = CSC // 16


def _sc_tail_body(embn, dstm, batchi, cent_out, cb_out,
                  labels_sh,
                  dstv, labels_v, batch_v, hist, cent_local, chunk, cbb):
    c = lax.axis_index("c")
    s = lax.axis_index("s")
    lanes = lax.iota(jnp.int32, 16)
    zeros16f = jnp.zeros((16,), jnp.float32)

    @pl.when(c == 0)
    def _core0():
        lo = s * LO_STEP

        def _z(r, _):
            for j in range(D // 16):
                chunk[r, pl.ds(16 * j, 16)] = zeros16f
            return 0
        lax.fori_loop(0, CSC, _z, 0)

        def _zo(i, _):
            pltpu.sync_copy(chunk, cent_out.at[pl.ds(lo + i * CSC, CSC)])
            return 0
        lax.fori_loop(0, LO_STEP // CSC, _zo, 0)

        def _zc(r, _):
            for j in range(D // 16):
                cent_local[r, pl.ds(16 * j, 16)] = zeros16f
            return 0
        lax.fori_loop(0, LO_STEP, _zc, 0)

        def _zh(g, _):
            hist[pl.ds(g * 16, 16)] = jnp.zeros((16,), jnp.int32)
            return 0
        lax.fori_loop(0, G, _zh, 0)

        @pl.when(s == 0)
        def _tile0():
            pltpu.sync_copy(dstm, dstv)

            def _init(g, _):
                labels_v[pl.ds(g * 16, 16)] = g * 16 + lanes
                return 0
            lax.fori_loop(0, G, _init, 0)

            def _sweep(it, _):
                def _mm(g, _2):
                    rows = g * 16 + lanes
                    lab = labels_v[pl.ds(g * 16, 16)]
                    acc = lab
                    flat = rows * K
                    for k in range(K):
                        dk = plsc.load_gather(dstv, [flat + k])
                        lv = plsc.load_gather(labels_v, [dk])
                        acc = jnp.minimum(acc, lv)
                        plsc.store_scatter(labels_v, [dk],
                                           jnp.minimum(lv, lab))
                    labels_v[pl.ds(g * 16, 16)] = acc
                    return 0
                lax.fori_loop(0, G, _mm, 0)

                def _jump(g, _2):
                    lab = labels_v[pl.ds(g * 16, 16)]
                    labels_v[pl.ds(g * 16, 16)] = plsc.load_gather(
                        labels_v, [lab])
                    return 0
                lax.fori_loop(0, G, _jump, 0)
                return 0
            lax.fori_loop(0, CC_ITERS, _sweep, 0)

            pltpu.sync_copy(labels_v, labels_sh)

        plsc.subcore_barrier()

        pltpu.sync_copy(labels_sh, labels_v)
        pltpu.sync_copy(batchi, batch_v)

        def _scan(ci, _):
            pltpu.sync_copy(embn.at[pl.ds(ci * CSC, CSC)], chunk)
            for gq in range(GC):
                lab16 = labels_v[pl.ds(ci * CSC + gq * 16, 16)]
                inr = jnp.where((lab16 >= lo) & (lab16 < lo + LO_STEP),
                                jnp.int32(1), jnp.int32(0))
                for r in range(16):
                    li = jnp.full((16,), lab16[r], jnp.int32)
                    hv = plsc.load_gather(hist, [li])
                    plsc.store_scatter(hist, [li], hv + 1)

                    @pl.when(inr[r] == 1)
                    def _add(lab16=lab16, r=r, gq=gq):
                        lloc = lab16[r] - lo
                        for j in range(D // 16):
                            cent_local[lloc, pl.ds(16 * j, 16)] = (
                                cent_local[lloc, pl.ds(16 * j, 16)]
                                + chunk[gq * 16 + r, pl.ds(16 * j, 16)])
            return 0
        lax.fori_loop(0, NCH, _scan, 0)

        def _basef(g, b):
            p16 = g * 16 + lanes
            vr = jnp.where((labels_v[pl.ds(g * 16, 16)] == p16)
                           & (hist[pl.ds(g * 16, 16)] >= MIN_SIZE),
                           jnp.int32(1), jnp.int32(0))
            return b + jnp.sum(vr)
        base = lax.fori_loop(0, s * (LO_STEP // 16), _basef, jnp.int32(0))

        def _place(g, rk):
            p16 = lo + g * 16 + lanes
            vr = jnp.where((labels_v[pl.ds(lo + g * 16, 16)] == p16)
                           & (hist[pl.ds(lo + g * 16, 16)] >= MIN_SIZE),
                           jnp.int32(1), jnp.int32(0))
            for r in range(16):

                @pl.when(vr[r] == 1)
                def _row(g=g, r=r, rk=rk, vr=vr):
                    rk2 = rk + jnp.sum(jnp.where(lanes < r, vr, 0))
                    pltpu.sync_copy(
                        cent_local.at[pl.ds(g * 16 + r, 1)],
                        cent_out.at[pl.ds(base + rk2, 1)])
            return rk + jnp.sum(vr)
        lax.fori_loop(0, LO_STEP // 16, _place, jnp.int32(0))

        @pl.when(s == 0)
        def _cb():
            def _rh(g, rooth):
                p16 = g * 16 + lanes
                vr = jnp.where((labels_v[pl.ds(g * 16, 16)] == p16)
                               & (hist[pl.ds(g * 16, 16)] >= MIN_SIZE),
                               jnp.int32(1), jnp.int32(0))
                bat16 = batch_v[pl.ds(g * 16, 16)]
                for r in range(16):
                    rooth = rooth + jnp.where(
                        (lanes == bat16[r]) & (vr[r] == 1),
                        jnp.int32(1), jnp.int32(0))
                return rooth
            rooth = lax.fori_loop(0, G, _rh, jnp.zeros((16,), jnp.int32))
            cum = plsc.cumsum(rooth)

            def _fill(o, _):
                def _grp(g, _2):
                    j16 = o * 256 + g * 16 + lanes
                    cb16 = jnp.zeros((16,), jnp.int32)
                    for b in range(NUM_BATCH):
                        cb16 = cb16 + jnp.where(j16 >= cum[b],
                                                jnp.int32(1), jnp.int32(0))
                    cbb[pl.ds(g * 16, 16)] = cb16
                    return 0
                lax.fori_loop(0, 16, _grp, 0)
                pltpu.sync_copy(cbb, cb_out.at[pl.ds(o * 256, 256)])
                return 0
            lax.fori_loop(0, 16, _fill, 0)


def _sc_tail(embn, dst, batch):
    mesh = plsc.VectorSubcoreMesh(core_axis_name="c", subcore_axis_name="s")
    f = pl.kernel(
        _sc_tail_body,
        out_type=[
            jax.ShapeDtypeStruct((N, D), jnp.float32),
            jax.ShapeDtypeStruct((N,), jnp.int32),
        ],
        mesh=mesh,
        compiler_params=pltpu.CompilerParams(needs_layout_passes=False),
        scratch_types=[
            pltpu.VMEM_SHARED((N,), jnp.int32),
            pltpu.VMEM((N * K,), jnp.int32),
            pltpu.VMEM((N,), jnp.int32),
            pltpu.VMEM((N,), jnp.int32),
            pltpu.VMEM((N,), jnp.int32),
            pltpu.VMEM((LO_STEP, D), jnp.float32),
            pltpu.VMEM((CSC, D), jnp.float32),
            pltpu.VMEM((256,), jnp.int32),
        ],
    )
    return f(embn, dst, batch)


def kernel(emb, batch):
    embn = _normalize(emb)
    dst = _topk_edges(embn, batch)
    dst_flat = dst[:, :K].reshape(N * K)
    cent_raw, cb_sorted = _sc_tail(embn, dst_flat, batch)
    centroids = _renorm(cent_raw)
    return centroids, cb_sorted


# speedup vs baseline: 22.0761x; 1.3171x over previous
import jax
import jax.numpy as jnp
from jax import lax
from jax.experimental import pallas as pl
from jax.experimental.pallas import tpu as pltpu
from jax.experimental.pallas import tpu_sc as plsc

N = 4096
D = 256
K = 10
KPAD = 16
NUM_BATCH = 8
MIN_SIZE = 10
CC_ITERS = 25

ROWS_BLK = 256
NT = 16
G = N // 16


def _normalize_body(emb_ref, out_ref):
    x = emb_ref[...]
    nrm = jnp.sqrt(jnp.sum(x * x, axis=1, keepdims=True))
    out_ref[...] = x / (nrm + 1e-12)


def _normalize(emb):
    return pl.pallas_call(
        _normalize_body,
        grid=(N // ROWS_BLK,),
        in_specs=[pl.BlockSpec((ROWS_BLK, D), lambda i: (i, 0))],
        out_specs=pl.BlockSpec((ROWS_BLK, D), lambda i: (i, 0)),
        out_shape=jax.ShapeDtypeStruct((N, D), jnp.float32),
    )(emb)


def _topk_body(a_ref, full_ref, bcol_ref, brow_ref, dst_ref):
    r0 = pl.program_id(0) * ROWS_BLK
    a = a_ref[...]
    full = full_ref[...]
    sim = lax.dot_general(a, full, (((1,), (1,)), ((), ())),
                          preferred_element_type=jnp.float32)
    col = lax.broadcasted_iota(jnp.int32, (ROWS_BLK, N), 1)
    rowg = r0 + lax.broadcasted_iota(jnp.int32, (ROWS_BLK, N), 0)
    same = bcol_ref[...] == brow_ref[...]
    neg = jnp.float32(-jnp.inf)
    masked = jnp.where(same & (col != rowg), sim, neg)

    selfc = r0 + lax.broadcasted_iota(jnp.int32, (ROWS_BLK, KPAD), 0)
    dstblk = selfc
    col16 = lax.broadcasted_iota(jnp.int32, (ROWS_BLK, KPAD), 1)
    for k in range(K):
        m = jnp.max(masked, axis=1, keepdims=True)
        pick = jnp.min(jnp.where(masked == m, col, N), axis=1,
                       keepdims=True)
        hit = col == pick
        t = jnp.max(jnp.where(hit, sim, neg), axis=1, keepdims=True)
        masked = jnp.where(hit, neg, masked)
        dstblk = jnp.where((col16 == k) & (t >= 0.0), pick, dstblk)
    dst_ref[...] = dstblk


def _topk_edges(embn, batch):
    return pl.pallas_call(
        _topk_body,
        grid=(N // ROWS_BLK,),
        in_specs=[
            pl.BlockSpec((ROWS_BLK, D), lambda i: (i, 0)),
            pl.BlockSpec((N, D), lambda i: (0, 0)),
            pl.BlockSpec((ROWS_BLK, 1), lambda i: (i, 0)),
            pl.BlockSpec((1, N), lambda i: (0, 0)),
        ],
        out_specs=pl.BlockSpec((ROWS_BLK, KPAD), lambda i: (i, 0)),
        out_shape=jax.ShapeDtypeStruct((N, KPAD), jnp.int32),
    )(embn, embn, batch.reshape(N, 1), batch.reshape(1, N))


def _renorm_body(c_ref, out_ref):
    x = c_ref[...]
    cn = jnp.sqrt(jnp.sum(x * x, axis=1, keepdims=True))
    out_ref[...] = x / jnp.where(cn > 0, cn, 1.0)


def _renorm(cent):
    return pl.pallas_call(
        _renorm_body,
        grid=(N // ROWS_BLK,),
        in_specs=[pl.BlockSpec((ROWS_BLK, D), lambda i: (i, 0))],
        out_specs=pl.BlockSpec((ROWS_BLK, D), lambda i: (i, 0)),
        out_shape=jax.ShapeDtypeStruct((N, D), jnp.float32),
    )(cent)


LO_STEP = N // NT
CSC = 32
NCH = N // CSC
GC = CSC // 16


def _sc_tail_body(embn, dstm, batchi, cent_out, cb_out,
                  labels_sh,
                  dstv, labels_v, batch_v, hist, cent_local, chunk, cbb):
    c = lax.axis_index("c")
    s = lax.axis_index("s")
    lanes = lax.iota(jnp.int32, 16)
    zeros16f = jnp.zeros((16,), jnp.float32)

    @pl.when(c == 0)
    def _core0():
        lo = s * LO_STEP

        def _z(r, _):
            for j in range(D // 16):
                chunk[r, pl.ds(16 * j, 16)] = zeros16f
            return 0
        lax.fori_loop(0, CSC, _z, 0)

        def _zo(i, _):
            pltpu.sync_copy(chunk, cent_out.at[pl.ds(lo + i * CSC, CSC)])
            return 0
        lax.fori_loop(0, LO_STEP // CSC, _zo, 0)

        def _zc(r, _):
            for j in range(D // 16):
                cent_local[r, pl.ds(16 * j, 16)] = zeros16f
            return 0
        lax.fori_loop(0, LO_STEP, _zc, 0)

        def _zh(g, _):
            hist[pl.ds(g * 16, 16)] = jnp.zeros((16,), jnp.int32)
            return 0
        lax.fori_loop(0, G, _zh, 0)

        @pl.when(s == 0)
        def _tile0():
            pltpu.sync_copy(dstm, dstv)

            def _init(g, _):
                labels_v[pl.ds(g * 16, 16)] = g * 16 + lanes
                return 0
            lax.fori_loop(0, G, _init, 0)

            def _sweep(it, _):
                def _mm(g, _2):
                    ga, gb = g, g + G // 2
                    rows_a = ga * 16 + lanes
                    rows_b = gb * 16 + lanes
                    lab_a = labels_v[pl.ds(ga * 16, 16)]
                    lab_b = labels_v[pl.ds(gb * 16, 16)]
                    acc_a, acc_b = lab_a, lab_b
                    flat_a = rows_a * K
                    flat_b = rows_b * K
                    for k in range(K):
                        dka = plsc.load_gather(dstv, [flat_a + k])
                        dkb = plsc.load_gather(dstv, [flat_b + k])
                        lva = plsc.load_gather(labels_v, [dka])
                        lvb = plsc.load_gather(labels_v, [dkb])
                        acc_a = jnp.minimum(acc_a, lva)
                        acc_b = jnp.minimum(acc_b, lvb)
                        plsc.store_scatter(labels_v, [dka],
                                           jnp.minimum(lva, lab_a))
                        plsc.store_scatter(labels_v, [dkb],
                                           jnp.minimum(lvb, lab_b))
                    labels_v[pl.ds(ga * 16, 16)] = acc_a
                    labels_v[pl.ds(gb * 16, 16)] = acc_b
                    return 0
                lax.fori_loop(0, G // 2, _mm, 0)

                def _jump(g, _2):
                    lab_a = labels_v[pl.ds(g * 32, 16)]
                    lab_b = labels_v[pl.ds(g * 32 + 16, 16)]
                    ja = plsc.load_gather(labels_v, [lab_a])
                    jb = plsc.load_gather(labels_v, [lab_b])
                    labels_v[pl.ds(g * 32, 16)] = ja
                    labels_v[pl.ds(g * 32 + 16, 16)] = jb
                    return 0
                lax.fori_loop(0, G // 2, _jump, 0)
                return 0
            lax.fori_loop(0, CC_ITERS, _sweep, 0)

            pltpu.sync_copy(labels_v, labels_sh)

        plsc.subcore_barrier()

        pltpu.sync_copy(labels_sh, labels_v)
        pltpu.sync_copy(batchi, batch_v)

        def _scan(ci, _):
            pltpu.sync_copy(embn.at[pl.ds(ci * CSC, CSC)], chunk)
            for gq in range(GC):
                lab16 = labels_v[pl.ds(ci * CSC + gq * 16, 16)]
                inr = jnp.where((lab16 >= lo) & (lab16 < lo + LO_STEP),
                                jnp.int32(1), jnp.int32(0))
                for r in range(16):
                    li = jnp.full((16,), lab16[r], jnp.int32)
                    hv = plsc.load_gather(hist, [li])
                    plsc.store_scatter(hist, [li], hv + 1)

                    @pl.when(inr[r] == 1)
                    def _add(lab16=lab16, r=r, gq=gq):
                        lloc = lab16[r] - lo
                        for j in range(D // 16):
                            cent_local[lloc, pl.ds(16 * j, 16)] = (
                                cent_local[lloc, pl.ds(16 * j, 16)]
                                + chunk[gq * 16 + r, pl.ds(16 * j, 16)])
            return 0
        lax.fori_loop(0, NCH, _scan, 0)

        def _basef(g, b):
            p16 = g * 16 + lanes
            vr = jnp.where((labels_v[pl.ds(g * 16, 16)] == p16)
                           & (hist[pl.ds(g * 16, 16)] >= MIN_SIZE),
                           jnp.int32(1), jnp.int32(0))
            return b + jnp.sum(vr)
        base = lax.fori_loop(0, s * (LO_STEP // 16), _basef, jnp.int32(0))

        def _place(g, rk):
            p16 = lo + g * 16 + lanes
            vr = jnp.where((labels_v[pl.ds(lo + g * 16, 16)] == p16)
                           & (hist[pl.ds(lo + g * 16, 16)] >= MIN_SIZE),
                           jnp.int32(1), jnp.int32(0))
            for r in range(16):

                @pl.when(vr[r] == 1)
                def _row(g=g, r=r, rk=rk, vr=vr):
                    rk2 = rk + jnp.sum(jnp.where(lanes < r, vr, 0))
                    pltpu.sync_copy(
                        cent_local.at[pl.ds(g * 16 + r, 1)],
                        cent_out.at[pl.ds(base + rk2, 1)])
            return rk + jnp.sum(vr)
        lax.fori_loop(0, LO_STEP // 16, _place, jnp.int32(0))

        @pl.when(s == 0)
        def _cb():
            def _rh(g, rooth):
                p16 = g * 16 + lanes
                vr = jnp.where((labels_v[pl.ds(g * 16, 16)] == p16)
                               & (hist[pl.ds(g * 16, 16)] >= MIN_SIZE),
                               jnp.int32(1), jnp.int32(0))
                bat16 = batch_v[pl.ds(g * 16, 16)]
                for r in range(16):
                    rooth = rooth + jnp.where(
                        (lanes == bat16[r]) & (vr[r] == 1),
                        jnp.int32(1), jnp.int32(0))
                return rooth
            rooth = lax.fori_loop(0, G, _rh, jnp.zeros((16,), jnp.int32))
            cum = plsc.cumsum(rooth)

            def _fill(o, _):
                def _grp(g, _2):
                    j16 = o * 256 + g * 16 + lanes
                    cb16 = jnp.zeros((16,), jnp.int32)
                    for b in range(NUM_BATCH):
                        cb16 = cb16 + jnp.where(j16 >= cum[b],
                                                jnp.int32(1), jnp.int32(0))
                    cbb[pl.ds(g * 16, 16)] = cb16
                    return 0
                lax.fori_loop(0, 16, _grp, 0)
                pltpu.sync_copy(cbb, cb_out.at[pl.ds(o * 256, 256)])
                return 0
            lax.fori_loop(0, 16, _fill, 0)


def _sc_tail(embn, dst, batch):
    mesh = plsc.VectorSubcoreMesh(core_axis_name="c", subcore_axis_name="s")
    f = pl.kernel(
        _sc_tail_body,
        out_type=[
            jax.ShapeDtypeStruct((N, D), jnp.float32),
            jax.ShapeDtypeStruct((N,), jnp.int32),
        ],
        mesh=mesh,
        compiler_params=pltpu.CompilerParams(needs_layout_passes=False),
        scratch_types=[
            pltpu.VMEM_SHARED((N,), jnp.int32),
            pltpu.VMEM((N * K,), jnp.int32),
            pltpu.VMEM((N,), jnp.int32),
            pltpu.VMEM((N,), jnp.int32),
            pltpu.VMEM((N,), jnp.int32),
            pltpu.VMEM((LO_STEP, D), jnp.float32),
            pltpu.VMEM((CSC, D), jnp.float32),
            pltpu.VMEM((256,), jnp.int32),
        ],
    )
    return f(embn, dst, batch)


def kernel(emb, batch):
    embn = _normalize(emb)
    dst = _topk_edges(embn, batch)
    dst_flat = dst[:, :K].reshape(N * K)
    cent_raw, cb_sorted = _sc_tail(embn, dst_flat, batch)
    centroids = _renorm(cent_raw)
    return centroids, cb_sorted


# speedup vs baseline: 25.5238x; 1.1562x over previous
import jax
import jax.numpy as jnp
from jax import lax
from jax.experimental import pallas as pl
from jax.experimental.pallas import tpu as pltpu
from jax.experimental.pallas import tpu_sc as plsc

N = 4096
D = 256
K = 10
KPAD = 16
NUM_BATCH = 8
MIN_SIZE = 10
CC_ITERS = 25

ROWS_BLK = 256
NT = 16
G = N // 16


def _normalize_body(emb_ref, out_ref):
    x = emb_ref[...]
    nrm = jnp.sqrt(jnp.sum(x * x, axis=1, keepdims=True))
    out_ref[...] = x / (nrm + 1e-12)


def _normalize(emb):
    return pl.pallas_call(
        _normalize_body,
        grid=(N // ROWS_BLK,),
        in_specs=[pl.BlockSpec((ROWS_BLK, D), lambda i: (i, 0))],
        out_specs=pl.BlockSpec((ROWS_BLK, D), lambda i: (i, 0)),
        out_shape=jax.ShapeDtypeStruct((N, D), jnp.float32),
    )(emb)


def _topk_body(a_ref, full_ref, bcol_ref, brow_ref, dst_ref):
    r0 = pl.program_id(0) * ROWS_BLK
    a = a_ref[...]
    full = full_ref[...]
    sim = lax.dot_general(a, full, (((1,), (1,)), ((), ())),
                          preferred_element_type=jnp.float32)
    col = lax.broadcasted_iota(jnp.int32, (ROWS_BLK, N), 1)
    rowg = r0 + lax.broadcasted_iota(jnp.int32, (ROWS_BLK, N), 0)
    same = bcol_ref[...] == brow_ref[...]
    neg = jnp.float32(-jnp.inf)
    masked = jnp.where(same & (col != rowg), sim, neg)

    selfc = r0 + lax.broadcasted_iota(jnp.int32, (ROWS_BLK, KPAD), 0)
    dstblk = selfc
    col16 = lax.broadcasted_iota(jnp.int32, (ROWS_BLK, KPAD), 1)
    for k in range(K):
        m = jnp.max(masked, axis=1, keepdims=True)
        pick = jnp.min(jnp.where(masked == m, col, N), axis=1,
                       keepdims=True)
        hit = col == pick
        t = jnp.max(jnp.where(hit, sim, neg), axis=1, keepdims=True)
        masked = jnp.where(hit, neg, masked)
        dstblk = jnp.where((col16 == k) & (t >= 0.0), pick, dstblk)
    dst_ref[...] = dstblk


def _topk_edges(embn, batch):
    return pl.pallas_call(
        _topk_body,
        grid=(N // ROWS_BLK,),
        in_specs=[
            pl.BlockSpec((ROWS_BLK, D), lambda i: (i, 0)),
            pl.BlockSpec((N, D), lambda i: (0, 0)),
            pl.BlockSpec((ROWS_BLK, 1), lambda i: (i, 0)),
            pl.BlockSpec((1, N), lambda i: (0, 0)),
        ],
        out_specs=pl.BlockSpec((ROWS_BLK, KPAD), lambda i: (i, 0)),
        out_shape=jax.ShapeDtypeStruct((N, KPAD), jnp.int32),
    )(embn, embn, batch.reshape(N, 1), batch.reshape(1, N))


def _renorm_body(c_ref, out_ref):
    x = c_ref[...]
    cn = jnp.sqrt(jnp.sum(x * x, axis=1, keepdims=True))
    out_ref[...] = x / jnp.where(cn > 0, cn, 1.0)


def _renorm(cent):
    return pl.pallas_call(
        _renorm_body,
        grid=(N // ROWS_BLK,),
        in_specs=[pl.BlockSpec((ROWS_BLK, D), lambda i: (i, 0))],
        out_specs=pl.BlockSpec((ROWS_BLK, D), lambda i: (i, 0)),
        out_shape=jax.ShapeDtypeStruct((N, D), jnp.float32),
    )(cent)


LO_STEP = N // NT
CSC = 32
NCH = N // CSC
GC = CSC // 16


def _sc_tail_body(embn, dstm, batchi, cent_out, cb_out,
                  labels_sh,
                  dstv, labels_v, batch_v, hist, cent_local, chunk, cbb):
    c = lax.axis_index("c")
    s = lax.axis_index("s")
    lanes = lax.iota(jnp.int32, 16)
    zeros16f = jnp.zeros((16,), jnp.float32)

    @pl.when(c == 0)
    def _core0():
        lo = s * LO_STEP

        def _z(r, _):
            for j in range(D // 16):
                chunk[r, pl.ds(16 * j, 16)] = zeros16f
            return 0
        lax.fori_loop(0, CSC, _z, 0)

        def _zo(i, _):
            pltpu.sync_copy(chunk, cent_out.at[pl.ds(lo + i * CSC, CSC)])
            return 0
        lax.fori_loop(0, LO_STEP // CSC, _zo, 0)

        def _zc(r, _):
            for j in range(D // 16):
                cent_local[r, pl.ds(16 * j, 16)] = zeros16f
            return 0
        lax.fori_loop(0, LO_STEP, _zc, 0)

        def _zh(g, _):
            hist[pl.ds(g * 16, 16)] = jnp.zeros((16,), jnp.int32)
            return 0
        lax.fori_loop(0, G, _zh, 0)

        @pl.when(s == 0)
        def _tile0():
            pltpu.sync_copy(dstm, dstv)

            def _init(g, _):
                labels_v[pl.ds(g * 16, 16)] = g * 16 + lanes
                return 0
            lax.fori_loop(0, G, _init, 0)

            def _sweep(it, _):
                NW = 4

                def _mm(g, _2):
                    gs = [g + w * (G // NW) for w in range(NW)]
                    labs = [labels_v[pl.ds(gg * 16, 16)] for gg in gs]
                    accs = list(labs)
                    flats = [(gg * 16 + lanes) * K for gg in gs]
                    for k in range(K):
                        dks = [plsc.load_gather(dstv, [f + k])
                               for f in flats]
                        lvs = [plsc.load_gather(labels_v, [dk])
                               for dk in dks]
                        for w in range(NW):
                            accs[w] = jnp.minimum(accs[w], lvs[w])
                            plsc.store_scatter(labels_v, [dks[w]],
                                               jnp.minimum(lvs[w], labs[w]))
                    for w in range(NW):
                        labels_v[pl.ds(gs[w] * 16, 16)] = accs[w]
                    return 0
                lax.fori_loop(0, G // NW, _mm, 0)

                def _jump(g, _2):
                    labs = [labels_v[pl.ds(g * 16 * NW + w * 16, 16)]
                            for w in range(NW)]
                    js = [plsc.load_gather(labels_v, [lb]) for lb in labs]
                    for w in range(NW):
                        labels_v[pl.ds(g * 16 * NW + w * 16, 16)] = js[w]
                    return 0
                lax.fori_loop(0, G // NW, _jump, 0)
                return 0
            lax.fori_loop(0, CC_ITERS, _sweep, 0)

            pltpu.sync_copy(labels_v, labels_sh)

        plsc.subcore_barrier()

        pltpu.sync_copy(labels_sh, labels_v)
        pltpu.sync_copy(batchi, batch_v)

        def _scan(ci, _):
            pltpu.sync_copy(embn.at[pl.ds(ci * CSC, CSC)], chunk)
            for gq in range(GC):
                lab16 = labels_v[pl.ds(ci * CSC + gq * 16, 16)]
                inr = jnp.where((lab16 >= lo) & (lab16 < lo + LO_STEP),
                                jnp.int32(1), jnp.int32(0))
                for r in range(16):
                    li = jnp.full((16,), lab16[r], jnp.int32)
                    hv = plsc.load_gather(hist, [li])
                    plsc.store_scatter(hist, [li], hv + 1)

                    @pl.when(inr[r] == 1)
                    def _add(lab16=lab16, r=r, gq=gq):
                        lloc = lab16[r] - lo
                        for j in range(D // 16):
                            cent_local[lloc, pl.ds(16 * j, 16)] = (
                                cent_local[lloc, pl.ds(16 * j, 16)]
                                + chunk[gq * 16 + r, pl.ds(16 * j, 16)])
            return 0
        lax.fori_loop(0, NCH, _scan, 0)

        def _basef(g, b):
            p16 = g * 16 + lanes
            vr = jnp.where((labels_v[pl.ds(g * 16, 16)] == p16)
                           & (hist[pl.ds(g * 16, 16)] >= MIN_SIZE),
                           jnp.int32(1), jnp.int32(0))
            return b + jnp.sum(vr)
        base = lax.fori_loop(0, s * (LO_STEP // 16), _basef, jnp.int32(0))

        def _place(g, rk):
            p16 = lo + g * 16 + lanes
            vr = jnp.where((labels_v[pl.ds(lo + g * 16, 16)] == p16)
                           & (hist[pl.ds(lo + g * 16, 16)] >= MIN_SIZE),
                           jnp.int32(1), jnp.int32(0))
            for r in range(16):

                @pl.when(vr[r] == 1)
                def _row(g=g, r=r, rk=rk, vr=vr):
                    rk2 = rk + jnp.sum(jnp.where(lanes < r, vr, 0))
                    pltpu.sync_copy(
                        cent_local.at[pl.ds(g * 16 + r, 1)],
                        cent_out.at[pl.ds(base + rk2, 1)])
            return rk + jnp.sum(vr)
        lax.fori_loop(0, LO_STEP // 16, _place, jnp.int32(0))

        @pl.when(s == 0)
        def _cb():
            def _rh(g, rooth):
                p16 = g * 16 + lanes
                vr = jnp.where((labels_v[pl.ds(g * 16, 16)] == p16)
                               & (hist[pl.ds(g * 16, 16)] >= MIN_SIZE),
                               jnp.int32(1), jnp.int32(0))
                bat16 = batch_v[pl.ds(g * 16, 16)]
                for r in range(16):
                    rooth = rooth + jnp.where(
                        (lanes == bat16[r]) & (vr[r] == 1),
                        jnp.int32(1), jnp.int32(0))
                return rooth
            rooth = lax.fori_loop(0, G, _rh, jnp.zeros((16,), jnp.int32))
            cum = plsc.cumsum(rooth)

            def _fill(o, _):
                def _grp(g, _2):
                    j16 = o * 256 + g * 16 + lanes
                    cb16 = jnp.zeros((16,), jnp.int32)
                    for b in range(NUM_BATCH):
                        cb16 = cb16 + jnp.where(j16 >= cum[b],
                                                jnp.int32(1), jnp.int32(0))
                    cbb[pl.ds(g * 16, 16)] = cb16
                    return 0
                lax.fori_loop(0, 16, _grp, 0)
                pltpu.sync_copy(cbb, cb_out.at[pl.ds(o * 256, 256)])
                return 0
            lax.fori_loop(0, 16, _fill, 0)


def _sc_tail(embn, dst, batch):
    mesh = plsc.VectorSubcoreMesh(core_axis_name="c", subcore_axis_name="s")
    f = pl.kernel(
        _sc_tail_body,
        out_type=[
            jax.ShapeDtypeStruct((N, D), jnp.float32),
            jax.ShapeDtypeStruct((N,), jnp.int32),
        ],
        mesh=mesh,
        compiler_params=pltpu.CompilerParams(needs_layout_passes=False),
        scratch_types=[
            pltpu.VMEM_SHARED((N,), jnp.int32),
            pltpu.VMEM((N * K,), jnp.int32),
            pltpu.VMEM((N,), jnp.int32),
            pltpu.VMEM((N,), jnp.int32),
            pltpu.VMEM((N,), jnp.int32),
            pltpu.VMEM((LO_STEP, D), jnp.float32),
            pltpu.VMEM((CSC, D), jnp.float32),
            pltpu.VMEM((256,), jnp.int32),
        ],
    )
    return f(embn, dst, batch)


def kernel(emb, batch):
    embn = _normalize(emb)
    dst = _topk_edges(embn, batch)
    dst_flat = dst[:, :K].reshape(N * K)
    cent_raw, cb_sorted = _sc_tail(embn, dst_flat, batch)
    centroids = _renorm(cent_raw)
    return centroids, cb_sorted
